# baseline (speedup 1.0000x reference)
import functools

import jax
import jax.numpy as jnp
from jax import lax
from jax.experimental import pallas as pl
from jax.experimental.pallas import tpu as pltpu
from jax.experimental.pallas import tpu_sc as plsc

VOCAB = 30522
HIDDEN = 768
MAX_POS = 512
EPS = 1e-12
B, S = 128, 512

NC, NS = 2, 16
NW = NC * NS
P = 8
BP = B // P
TOK_P = BP * S
TOK_W = TOK_P // NW
GCH = 32
NGC = TOK_W // GCH
NBUF = 4

BS_S = 8
TC_GRID = S // BS_S


def _sc_gather_body(ids_ref, word_ref, tmp_ref, idx_all, bufs, gsems, wsems):
    wid = lax.axis_index("s") * NC + lax.axis_index("c")
    base = wid * TOK_W

    pltpu.sync_copy(ids_ref.at[pl.ds(wid * NGC, NGC)], idx_all)

    def gather(c):
        pltpu.async_copy(word_ref.at[idx_all.at[c]], bufs.at[c % NBUF],
                         gsems.at[c % NBUF])

    def wait_gather(c):
        pltpu.make_async_copy(word_ref.at[idx_all.at[c]], bufs.at[c % NBUF],
                              gsems.at[c % NBUF]).wait()

    def write(c):
        pltpu.async_copy(bufs.at[c % NBUF],
                         tmp_ref.at[pl.ds(base + c * GCH, GCH)],
                         wsems.at[c % NBUF])

    def wait_write(c):
        pltpu.make_async_copy(bufs.at[c % NBUF],
                              tmp_ref.at[pl.ds(base + c * GCH, GCH)],
                              wsems.at[c % NBUF]).wait()

    for c in range(NGC):
        if c >= NBUF - 1:
            wait_write(c - (NBUF - 1))
        gather(c)
        if c >= 1:
            wait_gather(c - 1)
            write(c - 1)
    wait_gather(NGC - 1)
    write(NGC - 1)
    for c in range(NGC - (NBUF - 1), NGC):
        wait_write(c)


def _make_sc_gather():
    mesh = plsc.VectorSubcoreMesh(core_axis_name="c", subcore_axis_name="s")
    return pl.kernel(
        _sc_gather_body,
        out_type=jax.ShapeDtypeStruct((TOK_P, HIDDEN), jnp.float32),
        mesh=mesh,
        compiler_params=pltpu.CompilerParams(needs_layout_passes=False),
        scratch_types=[
            pltpu.VMEM((NGC, GCH), jnp.int32),
            pltpu.VMEM((NBUF, GCH, HIDDEN), jnp.float32),
            pltpu.SemaphoreType.DMA((NBUF,)),
            pltpu.SemaphoreType.DMA((NBUF,)),
        ],
    )


def _tc_ln_body(tmp_ref, pos_ref, type_ref, gamma_ref, beta_ref, out_ref):
    x = tmp_ref[...]
    bias = pos_ref[...] + type_ref[...]
    y = x + bias[None, :, :]
    mean = jnp.mean(y, axis=-1, keepdims=True)
    var = jnp.mean(jnp.square(y - mean), axis=-1, keepdims=True)
    normed = (y - mean) * lax.rsqrt(var + jnp.float32(EPS))
    out_ref[...] = normed * gamma_ref[...][None, :, :] + beta_ref[...][None]


def _tc_ln_alias_body(out_in_ref, tmp_ref, pos_ref, type_ref, gamma_ref,
                      beta_ref, out_ref):
    del out_in_ref
    _tc_ln_body(tmp_ref, pos_ref, type_ref, gamma_ref, beta_ref, out_ref)


def _tc_specs(p):
    in_specs = [
        pl.BlockSpec((BP, BS_S, HIDDEN), lambda i: (0, i, 0)),
        pl.BlockSpec((BS_S, HIDDEN), lambda i: (i, 0)),
        pl.BlockSpec((1, HIDDEN), lambda i: (0, 0)),
        pl.BlockSpec((1, HIDDEN), lambda i: (0, 0)),
        pl.BlockSpec((1, HIDDEN), lambda i: (0, 0)),
    ]
    out_spec = pl.BlockSpec((BP, BS_S, HIDDEN), lambda i, p=p: (p, i, 0))
    return in_specs, out_spec


def _make_tc_ln(p, aliased):
    in_specs, out_spec = _tc_specs(p)
    if aliased:
        in_specs = [pl.BlockSpec(memory_space=pl.ANY)] + in_specs
    return pl.pallas_call(
        _tc_ln_alias_body if aliased else _tc_ln_body,
        grid=(TC_GRID,),
        in_specs=in_specs,
        out_specs=out_spec,
        out_shape=jax.ShapeDtypeStruct((B, S, HIDDEN), jnp.float32),
        input_output_aliases={0: 0} if aliased else {},
    )


@functools.partial(jax.jit, static_argnames=())
def kernel(input_ids, attention_mask, labels, word_emb, pos_emb, type_emb,
           ln_gamma, ln_beta):
    del attention_mask
    ids_rows = input_ids.reshape(-1, GCH)
    pos2 = pos_emb[:S]
    type2 = type_emb[0:1]
    gamma2 = ln_gamma.reshape(1, HIDDEN)
    beta2 = ln_beta.reshape(1, HIDDEN)
    sc_gather = _make_sc_gather()
    rows_per_phase = TOK_P // GCH
    out = None
    for p in range(P):
        ids_p = lax.slice_in_dim(ids_rows, p * rows_per_phase,
                                 (p + 1) * rows_per_phase, axis=0)
        tmp_p = sc_gather(ids_p, word_emb)
        tmp_p = tmp_p.reshape(BP, S, HIDDEN)
        if out is None:
            out = _make_tc_ln(p, False)(tmp_p, pos2, type2, gamma2, beta2)
        else:
            out = _make_tc_ln(p, True)(out, tmp_p, pos2, type2, gamma2,
                                       beta2)
    return out, labels


# speedup vs baseline: 1.5829x; 1.5829x over previous
import functools

import jax
import jax.numpy as jnp
from jax import lax
from jax.experimental import pallas as pl
from jax.experimental.pallas import tpu as pltpu
from jax.experimental.pallas import tpu_sc as plsc

VOCAB = 30522
HIDDEN = 768
MAX_POS = 512
EPS = 1e-12
B, S = 128, 512

NC, NS = 2, 16
NW = NC * NS
P = 4
BP = B // P
TOK_P = BP * S
TOK_W = TOK_P // NW
GCH = 32
NGC = TOK_W // GCH
NBUF = 4

BS_S = 16
TC_GRID = S // BS_S


def _sc_gather_body(ids_ref, word_ref, tmp_ref, idx_all, bufs, gsems, wsems):
    wid = lax.axis_index("s") * NC + lax.axis_index("c")
    base = wid * TOK_W

    pltpu.sync_copy(ids_ref.at[pl.ds(wid * NGC, NGC)], idx_all)

    def gather(c):
        pltpu.async_copy(word_ref.at[idx_all.at[c]], bufs.at[c % NBUF],
                         gsems.at[c % NBUF])

    def wait_gather(c):
        pltpu.make_async_copy(word_ref.at[idx_all.at[c]], bufs.at[c % NBUF],
                              gsems.at[c % NBUF]).wait()

    def write(c):
        pltpu.async_copy(bufs.at[c % NBUF],
                         tmp_ref.at[pl.ds(base + c * GCH, GCH)],
                         wsems.at[c % NBUF])

    def wait_write(c):
        pltpu.make_async_copy(bufs.at[c % NBUF],
                              tmp_ref.at[pl.ds(base + c * GCH, GCH)],
                              wsems.at[c % NBUF]).wait()

    for c in range(NGC):
        if c >= NBUF - 1:
            wait_write(c - (NBUF - 1))
        gather(c)
        if c >= 1:
            wait_gather(c - 1)
            write(c - 1)
    wait_gather(NGC - 1)
    write(NGC - 1)
    for c in range(NGC - (NBUF - 1), NGC):
        wait_write(c)


def _make_sc_gather():
    mesh = plsc.VectorSubcoreMesh(core_axis_name="c", subcore_axis_name="s")
    return pl.kernel(
        _sc_gather_body,
        out_type=jax.ShapeDtypeStruct((TOK_P, HIDDEN), jnp.float32),
        mesh=mesh,
        compiler_params=pltpu.CompilerParams(needs_layout_passes=False),
        scratch_types=[
            pltpu.VMEM((NGC, GCH), jnp.int32),
            pltpu.VMEM((NBUF, GCH, HIDDEN), jnp.float32),
            pltpu.SemaphoreType.DMA((NBUF,)),
            pltpu.SemaphoreType.DMA((NBUF,)),
        ],
    )


def _tc_ln_body(tmp_ref, pos_ref, type_ref, gamma_ref, beta_ref, out_ref):
    x = tmp_ref[...]
    bias = pos_ref[...] + type_ref[...]
    y = x + bias[None, :, :]
    mean = jnp.mean(y, axis=-1, keepdims=True)
    m2 = jnp.mean(jnp.square(y), axis=-1, keepdims=True)
    var = jnp.maximum(m2 - jnp.square(mean), jnp.float32(0.0))
    normed = (y - mean) * lax.rsqrt(var + jnp.float32(EPS))
    out_ref[...] = normed * gamma_ref[...][None, :, :] + beta_ref[...][None]


def _tc_ln_alias_body(out_in_ref, tmp_ref, pos_ref, type_ref, gamma_ref,
                      beta_ref, out_ref):
    del out_in_ref
    _tc_ln_body(tmp_ref, pos_ref, type_ref, gamma_ref, beta_ref, out_ref)


def _tc_specs(p):
    in_specs = [
        pl.BlockSpec((BP, BS_S, HIDDEN), lambda i: (0, i, 0)),
        pl.BlockSpec((BS_S, HIDDEN), lambda i: (i, 0)),
        pl.BlockSpec((1, HIDDEN), lambda i: (0, 0)),
        pl.BlockSpec((1, HIDDEN), lambda i: (0, 0)),
        pl.BlockSpec((1, HIDDEN), lambda i: (0, 0)),
    ]
    out_spec = pl.BlockSpec((BP, BS_S, HIDDEN), lambda i, p=p: (p, i, 0))
    return in_specs, out_spec


def _make_tc_ln(p, aliased):
    in_specs, out_spec = _tc_specs(p)
    if aliased:
        in_specs = [pl.BlockSpec(memory_space=pl.ANY)] + in_specs
    return pl.pallas_call(
        _tc_ln_alias_body if aliased else _tc_ln_body,
        grid=(TC_GRID,),
        in_specs=in_specs,
        out_specs=out_spec,
        out_shape=jax.ShapeDtypeStruct((B, S, HIDDEN), jnp.float32),
        input_output_aliases={0: 0} if aliased else {},
    )


@functools.partial(jax.jit, static_argnames=())
def kernel(input_ids, attention_mask, labels, word_emb, pos_emb, type_emb,
           ln_gamma, ln_beta):
    del attention_mask
    ids_rows = input_ids.reshape(-1, GCH)
    pos2 = pos_emb[:S]
    type2 = type_emb[0:1]
    gamma2 = ln_gamma.reshape(1, HIDDEN)
    beta2 = ln_beta.reshape(1, HIDDEN)
    sc_gather = _make_sc_gather()
    rows_per_phase = TOK_P // GCH
    out = None
    for p in range(P):
        ids_p = lax.slice_in_dim(ids_rows, p * rows_per_phase,
                                 (p + 1) * rows_per_phase, axis=0)
        tmp_p = sc_gather(ids_p, word_emb)
        tmp_p = tmp_p.reshape(BP, S, HIDDEN)
        if out is None:
            out = _make_tc_ln(p, False)(tmp_p, pos2, type2, gamma2, beta2)
        else:
            out = _make_tc_ln(p, True)(out, tmp_p, pos2, type2, gamma2,
                                       beta2)
    return out, labels


# speedup vs baseline: 1.6634x; 1.0508x over previous
import functools

import jax
import jax.numpy as jnp
from jax import lax
from jax.experimental import pallas as pl
from jax.experimental.pallas import tpu as pltpu
from jax.experimental.pallas import tpu_sc as plsc

VOCAB = 30522
HIDDEN = 768
MAX_POS = 512
EPS = 1e-12
B, S = 128, 512

NC, NS = 2, 16
NW = NC * NS
P = 4
BP = B // P
TOK_P = BP * S
TOK_W = TOK_P // NW
GCH = 32
NGC = TOK_W // GCH
NBUF = 4

BS_S = 32
TC_GRID = S // BS_S


def _sc_gather_body(ids_ref, word_ref, tmp_ref, idx_all, bufs, gsems, wsems):
    wid = lax.axis_index("s") * NC + lax.axis_index("c")
    base = wid * TOK_W

    pltpu.sync_copy(ids_ref.at[pl.ds(wid * NGC, NGC)], idx_all)

    def gather(c):
        pltpu.async_copy(word_ref.at[idx_all.at[c]], bufs.at[c % NBUF],
                         gsems.at[c % NBUF])

    def wait_gather(c):
        pltpu.make_async_copy(word_ref.at[idx_all.at[c]], bufs.at[c % NBUF],
                              gsems.at[c % NBUF]).wait()

    def write(c):
        pltpu.async_copy(bufs.at[c % NBUF],
                         tmp_ref.at[pl.ds(base + c * GCH, GCH)],
                         wsems.at[c % NBUF])

    def wait_write(c):
        pltpu.make_async_copy(bufs.at[c % NBUF],
                              tmp_ref.at[pl.ds(base + c * GCH, GCH)],
                              wsems.at[c % NBUF]).wait()

    for c in range(NGC):
        if c >= NBUF - 1:
            wait_write(c - (NBUF - 1))
        gather(c)
        if c >= 1:
            wait_gather(c - 1)
            write(c - 1)
    wait_gather(NGC - 1)
    write(NGC - 1)
    for c in range(NGC - (NBUF - 1), NGC):
        wait_write(c)


def _make_sc_gather():
    mesh = plsc.VectorSubcoreMesh(core_axis_name="c", subcore_axis_name="s")
    return pl.kernel(
        _sc_gather_body,
        out_type=jax.ShapeDtypeStruct((TOK_P, HIDDEN), jnp.float32),
        mesh=mesh,
        compiler_params=pltpu.CompilerParams(needs_layout_passes=False),
        scratch_types=[
            pltpu.VMEM((NGC, GCH), jnp.int32),
            pltpu.VMEM((NBUF, GCH, HIDDEN), jnp.float32),
            pltpu.SemaphoreType.DMA((NBUF,)),
            pltpu.SemaphoreType.DMA((NBUF,)),
        ],
    )


def _tc_ln_body(tmp_ref, pos_ref, type_ref, gamma_ref, beta_ref, out_ref):
    x = tmp_ref[...]
    bias = pos_ref[...] + type_ref[...]
    y = x + bias[None, :, :]
    mean = jnp.mean(y, axis=-1, keepdims=True)
    m2 = jnp.mean(jnp.square(y), axis=-1, keepdims=True)
    var = jnp.maximum(m2 - jnp.square(mean), jnp.float32(0.0))
    normed = (y - mean) * lax.rsqrt(var + jnp.float32(EPS))
    out_ref[...] = normed * gamma_ref[...][None, :, :] + beta_ref[...][None]


def _tc_ln_alias_body(out_in_ref, tmp_ref, pos_ref, type_ref, gamma_ref,
                      beta_ref, out_ref):
    del out_in_ref
    _tc_ln_body(tmp_ref, pos_ref, type_ref, gamma_ref, beta_ref, out_ref)


def _tc_specs(p):
    in_specs = [
        pl.BlockSpec((BP, BS_S, HIDDEN), lambda i: (0, i, 0)),
        pl.BlockSpec((BS_S, HIDDEN), lambda i: (i, 0)),
        pl.BlockSpec((1, HIDDEN), lambda i: (0, 0)),
        pl.BlockSpec((1, HIDDEN), lambda i: (0, 0)),
        pl.BlockSpec((1, HIDDEN), lambda i: (0, 0)),
    ]
    out_spec = pl.BlockSpec((BP, BS_S, HIDDEN), lambda i, p=p: (p, i, 0))
    return in_specs, out_spec


def _make_tc_ln(p, aliased):
    in_specs, out_spec = _tc_specs(p)
    if aliased:
        in_specs = [pl.BlockSpec(memory_space=pl.ANY)] + in_specs
    return pl.pallas_call(
        _tc_ln_alias_body if aliased else _tc_ln_body,
        grid=(TC_GRID,),
        in_specs=in_specs,
        out_specs=out_spec,
        out_shape=jax.ShapeDtypeStruct((B, S, HIDDEN), jnp.float32),
        input_output_aliases={0: 0} if aliased else {},
    )


@functools.partial(jax.jit, static_argnames=())
def kernel(input_ids, attention_mask, labels, word_emb, pos_emb, type_emb,
           ln_gamma, ln_beta):
    del attention_mask
    ids_rows = input_ids.reshape(-1, GCH)
    pos2 = pos_emb[:S]
    type2 = type_emb[0:1]
    gamma2 = ln_gamma.reshape(1, HIDDEN)
    beta2 = ln_beta.reshape(1, HIDDEN)
    sc_gather = _make_sc_gather()
    rows_per_phase = TOK_P // GCH
    out = None
    for p in range(P):
        ids_p = lax.slice_in_dim(ids_rows, p * rows_per_phase,
                                 (p + 1) * rows_per_phase, axis=0)
        tmp_p = sc_gather(ids_p, word_emb)
        tmp_p = tmp_p.reshape(BP, S, HIDDEN)
        if out is None:
            out = _make_tc_ln(p, False)(tmp_p, pos2, type2, gamma2, beta2)
        else:
            out = _make_tc_ln(p, True)(out, tmp_p, pos2, type2, gamma2,
                                       beta2)
    return out, labels


# speedup vs baseline: 1.6809x; 1.0105x over previous
import functools

import jax
import jax.numpy as jnp
from jax import lax
from jax.experimental import pallas as pl
from jax.experimental.pallas import tpu as pltpu
from jax.experimental.pallas import tpu_sc as plsc

VOCAB = 30522
HIDDEN = 768
MAX_POS = 512
EPS = 1e-12
B, S = 128, 512

NC, NS = 2, 16
NW = NC * NS
P = 4
BP = B // P
TOK_P = BP * S
TOK_W = TOK_P // NW
GCH = 32
NGC = TOK_W // GCH
NBUF = 4

BS_S = 64
TC_GRID = S // BS_S


def _sc_gather_body(ids_ref, word_ref, tmp_ref, idx_all, bufs, gsems, wsems):
    wid = lax.axis_index("s") * NC + lax.axis_index("c")
    base = wid * TOK_W

    pltpu.sync_copy(ids_ref.at[pl.ds(wid * NGC, NGC)], idx_all)

    def gather(c):
        pltpu.async_copy(word_ref.at[idx_all.at[c]], bufs.at[c % NBUF],
                         gsems.at[c % NBUF])

    def wait_gather(c):
        pltpu.make_async_copy(word_ref.at[idx_all.at[c]], bufs.at[c % NBUF],
                              gsems.at[c % NBUF]).wait()

    def write(c):
        pltpu.async_copy(bufs.at[c % NBUF],
                         tmp_ref.at[pl.ds(base + c * GCH, GCH)],
                         wsems.at[c % NBUF])

    def wait_write(c):
        pltpu.make_async_copy(bufs.at[c % NBUF],
                              tmp_ref.at[pl.ds(base + c * GCH, GCH)],
                              wsems.at[c % NBUF]).wait()

    for c in range(NGC):
        if c >= NBUF - 1:
            wait_write(c - (NBUF - 1))
        gather(c)
        if c >= 1:
            wait_gather(c - 1)
            write(c - 1)
    wait_gather(NGC - 1)
    write(NGC - 1)
    for c in range(NGC - (NBUF - 1), NGC):
        wait_write(c)


def _make_sc_gather():
    mesh = plsc.VectorSubcoreMesh(core_axis_name="c", subcore_axis_name="s")
    return pl.kernel(
        _sc_gather_body,
        out_type=jax.ShapeDtypeStruct((TOK_P, HIDDEN), jnp.float32),
        mesh=mesh,
        compiler_params=pltpu.CompilerParams(needs_layout_passes=False),
        scratch_types=[
            pltpu.VMEM((NGC, GCH), jnp.int32),
            pltpu.VMEM((NBUF, GCH, HIDDEN), jnp.float32),
            pltpu.SemaphoreType.DMA((NBUF,)),
            pltpu.SemaphoreType.DMA((NBUF,)),
        ],
    )


def _tc_ln_body(tmp_ref, pos_ref, type_ref, gamma_ref, beta_ref, out_ref):
    x = tmp_ref[...]
    bias = pos_ref[...] + type_ref[...]
    y = x + bias[None, :, :]
    mean = jnp.mean(y, axis=-1, keepdims=True)
    m2 = jnp.mean(jnp.square(y), axis=-1, keepdims=True)
    var = jnp.maximum(m2 - jnp.square(mean), jnp.float32(0.0))
    normed = (y - mean) * lax.rsqrt(var + jnp.float32(EPS))
    out_ref[...] = normed * gamma_ref[...][None, :, :] + beta_ref[...][None]


def _tc_ln_alias_body(out_in_ref, tmp_ref, pos_ref, type_ref, gamma_ref,
                      beta_ref, out_ref):
    del out_in_ref
    _tc_ln_body(tmp_ref, pos_ref, type_ref, gamma_ref, beta_ref, out_ref)


def _tc_specs(p):
    in_specs = [
        pl.BlockSpec((BP, BS_S, HIDDEN), lambda i: (0, i, 0)),
        pl.BlockSpec((BS_S, HIDDEN), lambda i: (i, 0)),
        pl.BlockSpec((1, HIDDEN), lambda i: (0, 0)),
        pl.BlockSpec((1, HIDDEN), lambda i: (0, 0)),
        pl.BlockSpec((1, HIDDEN), lambda i: (0, 0)),
    ]
    out_spec = pl.BlockSpec((BP, BS_S, HIDDEN), lambda i, p=p: (p, i, 0))
    return in_specs, out_spec


def _make_tc_ln(p, aliased):
    in_specs, out_spec = _tc_specs(p)
    if aliased:
        in_specs = [pl.BlockSpec(memory_space=pl.ANY)] + in_specs
    return pl.pallas_call(
        _tc_ln_alias_body if aliased else _tc_ln_body,
        grid=(TC_GRID,),
        in_specs=in_specs,
        out_specs=out_spec,
        out_shape=jax.ShapeDtypeStruct((B, S, HIDDEN), jnp.float32),
        input_output_aliases={0: 0} if aliased else {},
    )


@functools.partial(jax.jit, static_argnames=())
def kernel(input_ids, attention_mask, labels, word_emb, pos_emb, type_emb,
           ln_gamma, ln_beta):
    del attention_mask
    ids_rows = input_ids.reshape(-1, GCH)
    pos2 = pos_emb[:S]
    type2 = type_emb[0:1]
    gamma2 = ln_gamma.reshape(1, HIDDEN)
    beta2 = ln_beta.reshape(1, HIDDEN)
    sc_gather = _make_sc_gather()
    rows_per_phase = TOK_P // GCH
    out = None
    for p in range(P):
        ids_p = lax.slice_in_dim(ids_rows, p * rows_per_phase,
                                 (p + 1) * rows_per_phase, axis=0)
        tmp_p = sc_gather(ids_p, word_emb)
        tmp_p = tmp_p.reshape(BP, S, HIDDEN)
        if out is None:
            out = _make_tc_ln(p, False)(tmp_p, pos2, type2, gamma2, beta2)
        else:
            out = _make_tc_ln(p, True)(out, tmp_p, pos2, type2, gamma2,
                                       beta2)
    return out, labels
